# Initial kernel scaffold; baseline (speedup 1.0000x reference)
import jax
import jax.numpy as jnp
from jax.experimental import pallas as pl


def kernel(x, current_node_idx, edge_index, W_in, b_in, W1, b1, W2, b2, W_out, b_out):
    raise NotImplementedError("write your pallas kernel here")


# speedup vs baseline: 12.2387x; 12.2387x over previous
import functools

import jax
import jax.numpy as jnp
from jax import lax
from jax.experimental import pallas as pl
from jax.experimental.pallas import tpu as pltpu
from jax.experimental.pallas import tpu_sc as plsc

_N = 10000
_E = 320000
_D = 128
_Q = 1024

_NC = 2
_NS = 16
_CHUNK = 80

_STRIPE = 640
_LAST_OFF = _N - 512
_LAST = 512

_F32 = jnp.float32


def _mesh():
    return plsc.VectorSubcoreMesh(
        core_axis_name="c", subcore_axis_name="s",
        num_cores=_NC, num_subcores=_NS)


def _for_stripe(s, emit):
    @pl.when(s < _NS - 1)
    def _():
        emit(pl.multiple_of(s * _STRIPE, 8), _STRIPE)

    @pl.when(s == _NS - 1)
    def _():
        emit(_LAST_OFF, _LAST)


def _build_deg():
    ec = _E // _NS
    nch = ec // _CHUNK
    qc = _Q // _NS

    @functools.partial(
        pl.kernel,
        out_type=(jax.ShapeDtypeStruct((_N,), _F32),
                  jax.ShapeDtypeStruct((_Q,), _F32)),
        mesh=_mesh(),
        scratch_types=(
            pltpu.VMEM_SHARED((_N,), _F32),
            pltpu.VMEM((_STRIPE,), _F32),
            pltpu.VMEM((1, _CHUNK), jnp.int32),
            pltpu.VMEM((_CHUNK,), _F32),
            pltpu.VMEM((1, qc), jnp.int32),
            pltpu.VMEM((qc,), _F32),
            pltpu.SemaphoreType.DMA,
        ),
    )
    def deg_kernel(dst_hbm, q_hbm, deg_out, dq_out,
                   deg_sp, stage, didx, ones, qidx, dqv, sem):
        c = lax.axis_index("c")
        s = lax.axis_index("s")

        def zero_body(k, carry):
            stage[pl.ds(k * 16, 16)] = jnp.zeros((16,), _F32)
            return carry
        lax.fori_loop(0, _STRIPE // 16, zero_body, 0)
        for k in range(_CHUNK // 16):
            ones[pl.ds(k * 16, 16)] = jnp.full((16,), 1.0, _F32)

        def init(r0, nr):
            pltpu.sync_copy(stage.at[pl.ds(0, nr)], deg_sp.at[pl.ds(r0, nr)])
        _for_stripe(s, init)
        plsc.subcore_barrier()

        base = s * ec

        def edge_body(j, carry):
            off = pl.multiple_of(base + j * _CHUNK, 8)
            pltpu.sync_copy(dst_hbm.at[pl.ds(off, _CHUNK)], didx.at[0])
            pltpu.sync_copy(ones, deg_sp.at[didx.at[0]], add=True)
            return carry
        lax.fori_loop(0, nch, edge_body, 0)
        plsc.subcore_barrier()

        @pl.when(c == 0)
        def _():
            def wout(r0, nr):
                pltpu.sync_copy(deg_sp.at[pl.ds(r0, nr)],
                                stage.at[pl.ds(0, nr)])
                pltpu.sync_copy(stage.at[pl.ds(0, nr)],
                                deg_out.at[pl.ds(r0, nr)])
            _for_stripe(s, wout)
            qb = pl.multiple_of(s * qc, 8)
            pltpu.sync_copy(q_hbm.at[pl.ds(qb, qc)], qidx.at[0])
            pltpu.async_copy(deg_sp.at[qidx.at[0]], dqv, sem).wait()
            pltpu.sync_copy(dqv, dq_out.at[pl.ds(qb, qc)])

    return deg_kernel


def _build_seg(gather_q):
    ec = _E // _NC // _NS
    nch = ec // _CHUNK
    qc = _Q // _NS

    if gather_q:
        out_type = (jax.ShapeDtypeStruct((_NC, _Q, _D), _F32),
                    jax.ShapeDtypeStruct((_Q, _D), _F32))
        extra = (pltpu.VMEM((1, qc), jnp.int32),
                 pltpu.VMEM((qc, _D), _F32))
    else:
        out_type = jax.ShapeDtypeStruct((_NC, _N, _D), _F32)
        extra = ()

    @functools.partial(
        pl.kernel,
        out_type=out_type,
        mesh=_mesh(),
        scratch_types=(
            pltpu.VMEM_SHARED((_N, _D), _F32),
            pltpu.VMEM((1, _CHUNK), jnp.int32),
            pltpu.VMEM((1, _CHUNK), jnp.int32),
            pltpu.VMEM((_CHUNK, _D), _F32),
            pltpu.VMEM((128, _D), _F32),
            pltpu.SemaphoreType.DMA,
        ) + extra,
    )
    def seg_kernel(u_hbm, src_hbm, dst_hbm, *rest):
        if gather_q:
            (q_hbm, g_out, uq_out,
             acc, sidx, didx, rows, stage, sem, qidx, qrows) = rest
        else:
            s_out, acc, sidx, didx, rows, stage, sem = rest

        c = lax.axis_index("c")
        s = lax.axis_index("s")

        def init(r0, nr):
            def cp(k, carry):
                rr = pl.multiple_of(r0 + k * 128, 8)
                pltpu.sync_copy(u_hbm.at[pl.ds(rr, 128), :], stage)
                pltpu.sync_copy(stage, acc.at[pl.ds(rr, 128), :])
                return carry
            lax.fori_loop(0, nr // 128, cp, 0)
        _for_stripe(s, init)
        plsc.subcore_barrier()

        base = c * (_E // _NC) + s * ec

        def edge_body(j, carry):
            off = pl.multiple_of(base + j * _CHUNK, 8)
            pltpu.sync_copy(src_hbm.at[pl.ds(off, _CHUNK)], sidx.at[0])
            pltpu.sync_copy(dst_hbm.at[pl.ds(off, _CHUNK)], didx.at[0])
            pltpu.async_copy(u_hbm.at[sidx.at[0]], rows, sem).wait()
            pltpu.sync_copy(rows, acc.at[didx.at[0]], add=True)
            return carry
        lax.fori_loop(0, nch, edge_body, 0)
        plsc.subcore_barrier()

        if gather_q:
            qb = pl.multiple_of(s * qc, 8)
            pltpu.sync_copy(q_hbm.at[pl.ds(qb, qc)], qidx.at[0])
            pltpu.async_copy(acc.at[qidx.at[0]], qrows, sem).wait()
            pltpu.sync_copy(qrows, g_out.at[c, pl.ds(qb, qc), :])

            @pl.when(c == 1)
            def _():
                pltpu.async_copy(u_hbm.at[qidx.at[0]], qrows, sem).wait()
                pltpu.sync_copy(qrows, uq_out.at[pl.ds(qb, qc), :])
        else:
            def wout(r0, nr):
                def cp(k, carry):
                    rr = pl.multiple_of(r0 + k * 128, 8)
                    pltpu.sync_copy(acc.at[pl.ds(rr, 128), :], stage)
                    pltpu.sync_copy(stage, s_out.at[c, pl.ds(rr, 128), :])
                    return carry
                lax.fori_loop(0, nr // 128, cp, 0)
            _for_stripe(s, wout)

    return seg_kernel


_deg_call = _build_deg()
_seg1_call = _build_seg(gather_q=False)
_seg2_call = _build_seg(gather_q=True)


_BR = 1000


def _tc1(x, w_in, b_in, w1, deg):
    def body(x_ref, win_ref, bin_ref, w1_ref, deg_ref, out_ref):
        h = jnp.maximum(
            jnp.dot(x_ref[...], win_ref[...],
                    preferred_element_type=_F32) + bin_ref[...], 0.0)
        dinv = lax.rsqrt(deg_ref[...] + 1.0)
        out_ref[...] = jnp.dot(h, w1_ref[...],
                               preferred_element_type=_F32) * dinv

    return pl.pallas_call(
        body,
        grid=(_N // _BR,),
        in_specs=[pl.BlockSpec((_BR, _D), lambda i: (i, 0)),
                  pl.BlockSpec((_D, _D), lambda i: (0, 0)),
                  pl.BlockSpec((1, _D), lambda i: (0, 0)),
                  pl.BlockSpec((_D, _D), lambda i: (0, 0)),
                  pl.BlockSpec((_BR, 1), lambda i: (i, 0))],
        out_specs=pl.BlockSpec((_BR, _D), lambda i: (i, 0)),
        out_shape=jax.ShapeDtypeStruct((_N, _D), _F32),
    )(x, w_in, b_in, w1, deg)


def _tc2(sparts, u1, deg, b1, w2):
    def body(sp_ref, u1_ref, deg_ref, b1_ref, w2_ref, out_ref):
        dinv = lax.rsqrt(deg_ref[...] + 1.0)
        st = sp_ref[0] + sp_ref[1] - u1_ref[...]
        h = jnp.maximum(st * dinv + b1_ref[...], 0.0)
        out_ref[...] = jnp.dot(h, w2_ref[...],
                               preferred_element_type=_F32) * dinv

    return pl.pallas_call(
        body,
        grid=(_N // _BR,),
        in_specs=[pl.BlockSpec((_NC, _BR, _D), lambda i: (0, i, 0)),
                  pl.BlockSpec((_BR, _D), lambda i: (i, 0)),
                  pl.BlockSpec((_BR, 1), lambda i: (i, 0)),
                  pl.BlockSpec((1, _D), lambda i: (0, 0)),
                  pl.BlockSpec((_D, _D), lambda i: (0, 0))],
        out_specs=pl.BlockSpec((_BR, _D), lambda i: (i, 0)),
        out_shape=jax.ShapeDtypeStruct((_N, _D), _F32),
    )(sparts, u1, deg, b1, w2)


def _tc3(g, uq, dq, b2, w_out, b_out):
    def body(g_ref, uq_ref, dq_ref, b2_ref, wout_ref, bout_ref, out_ref):
        dinv = lax.rsqrt(dq_ref[...] + 1.0)
        st = g_ref[0] + g_ref[1] - uq_ref[...]
        h = jnp.maximum(st * dinv + b2_ref[...], 0.0)
        out_ref[...] = jnp.dot(h, wout_ref[...],
                               preferred_element_type=_F32) + bout_ref[...]

    return pl.pallas_call(
        body,
        grid=(1,),
        in_specs=[pl.BlockSpec((_NC, _Q, _D), lambda i: (0, 0, 0)),
                  pl.BlockSpec((_Q, _D), lambda i: (0, 0)),
                  pl.BlockSpec((_Q, 1), lambda i: (0, 0)),
                  pl.BlockSpec((1, _D), lambda i: (0, 0)),
                  pl.BlockSpec((_D, _D), lambda i: (0, 0)),
                  pl.BlockSpec((1, _D), lambda i: (0, 0))],
        out_specs=pl.BlockSpec((_Q, _D), lambda i: (0, 0)),
        out_shape=jax.ShapeDtypeStruct((_Q, _D), _F32),
    )(g, uq, dq, b2, w_out, b_out)


def kernel(x, current_node_idx, edge_index, W_in, b_in, W1, b1, W2, b2,
           W_out, b_out):
    src = edge_index[0].astype(jnp.int32)
    dst = edge_index[1].astype(jnp.int32)
    q = current_node_idx.astype(jnp.int32)

    deg, dq = _deg_call(dst, q)
    deg2 = deg.reshape(_N, 1)

    u1 = _tc1(x, W_in, b_in.reshape(1, _D), W1, deg2)
    sparts = _seg1_call(u1, src, dst)
    u2 = _tc2(sparts, u1, deg2, b1.reshape(1, _D), W2)
    g, uq = _seg2_call(u2, src, dst, q)
    return _tc3(g, uq, dq.reshape(_Q, 1), b2.reshape(1, _D),
                W_out, b_out.reshape(1, _D))


# speedup vs baseline: 24.5839x; 2.0087x over previous
import functools

import jax
import jax.numpy as jnp
from jax import lax
from jax.experimental import pallas as pl
from jax.experimental.pallas import tpu as pltpu
from jax.experimental.pallas import tpu_sc as plsc

_N = 10000
_E = 320000
_D = 128
_Q = 1024

_NC = 2
_NS = 16
_CHUNK = 80
_NCH = _E // _NC // _NS // _CHUNK
_QC = _Q // _NS

_STRIPE = 640
_LAST_OFF = _N - 512
_LAST = 512

_F32 = jnp.float32


def _mesh():
    return plsc.VectorSubcoreMesh(
        core_axis_name="c", subcore_axis_name="s",
        num_cores=_NC, num_subcores=_NS)


def _for_stripe(s, emit):
    @pl.when(s < _NS - 1)
    def _():
        emit(pl.multiple_of(s * _STRIPE, 8), _STRIPE)

    @pl.when(s == _NS - 1)
    def _():
        emit(_LAST_OFF, _LAST)


def _build_deg():
    @functools.partial(
        pl.kernel,
        out_type=(jax.ShapeDtypeStruct((_NC * _N,), _F32),
                  jax.ShapeDtypeStruct((_NC * _Q,), _F32)),
        mesh=_mesh(),
        scratch_types=(
            pltpu.VMEM_SHARED((_N,), _F32),
            pltpu.VMEM((_STRIPE,), _F32),
            pltpu.VMEM((_NCH * _CHUNK,), jnp.int32),
            pltpu.VMEM((_NCH, _CHUNK), jnp.int32),
            pltpu.VMEM((_CHUNK,), _F32),
            pltpu.VMEM((1, _QC), jnp.int32),
            pltpu.VMEM((_QC,), _F32),
            pltpu.SemaphoreType.DMA,
        ),
    )
    def deg_kernel(dst_hbm, q_hbm, deg_out, dq_out,
                   deg_sp, stage, dslab1, dslab, ones, qidx, dqv, sem):
        c = lax.axis_index("c")
        s = lax.axis_index("s")

        def zero_body(k, carry):
            stage[pl.ds(k * 16, 16)] = jnp.zeros((16,), _F32)
            return carry
        lax.fori_loop(0, _STRIPE // 16, zero_body, 0)
        for k in range(_CHUNK // 16):
            ones[pl.ds(k * 16, 16)] = jnp.full((16,), 1.0, _F32)

        base = pl.multiple_of((c * _NS + s) * (_NCH * _CHUNK), 8)
        pltpu.sync_copy(dst_hbm.at[pl.ds(base, _NCH * _CHUNK)], dslab1)

        def repack(j, carry):
            for k in range(_CHUNK // 16):
                dslab[j, pl.ds(k * 16, 16)] = (
                    dslab1[pl.ds(j * _CHUNK + k * 16, 16)])
            return carry
        lax.fori_loop(0, _NCH, repack, 0)

        def init(r0, nr):
            pltpu.sync_copy(stage.at[pl.ds(0, nr)], deg_sp.at[pl.ds(r0, nr)])
        _for_stripe(s, init)
        plsc.subcore_barrier()

        def edge_body(j, carry):
            pltpu.sync_copy(ones, deg_sp.at[dslab.at[j]], add=True)
            return carry
        lax.fori_loop(0, _NCH, edge_body, 0)
        plsc.subcore_barrier()

        def wout(r0, nr):
            pltpu.sync_copy(deg_sp.at[pl.ds(r0, nr)], stage.at[pl.ds(0, nr)])
            o0 = pl.multiple_of(c * _N + r0, 8)
            pltpu.sync_copy(stage.at[pl.ds(0, nr)],
                            deg_out.at[pl.ds(o0, nr)])
        _for_stripe(s, wout)
        qb = pl.multiple_of(s * _QC, 8)
        pltpu.sync_copy(q_hbm.at[pl.ds(qb, _QC)], qidx.at[0])
        pltpu.async_copy(deg_sp.at[qidx.at[0]], dqv, sem).wait()
        oq = pl.multiple_of(c * _Q + qb, 8)
        pltpu.sync_copy(dqv, dq_out.at[pl.ds(oq, _QC)])

    return deg_kernel


def _build_seg(gather_q):
    if gather_q:
        out_type = (jax.ShapeDtypeStruct((_NC, _Q, _D), _F32),
                    jax.ShapeDtypeStruct((_Q, _D), _F32))
        extra = (pltpu.VMEM((8, _QC // 8), jnp.int32),
                 pltpu.VMEM((_QC // 8, _D), _F32))
    else:
        out_type = jax.ShapeDtypeStruct((_NC, _N, _D), _F32)
        extra = ()

    _DB = 2000
    _DBR = _DB // _CHUNK

    @functools.partial(
        pl.kernel,
        out_type=out_type,
        mesh=_mesh(),
        scratch_types=(
            pltpu.VMEM_SHARED((_N, _D), _F32),
            pltpu.VMEM((_NCH * _CHUNK,), jnp.int32),
            pltpu.VMEM((_DB,), jnp.int32),
            pltpu.VMEM((_NCH, _CHUNK), jnp.int32),
            pltpu.VMEM((_CHUNK, _D), _F32),
            pltpu.VMEM((_CHUNK, _D), _F32),
            pltpu.SemaphoreType.DMA,
            pltpu.SemaphoreType.DMA,
            pltpu.SemaphoreType.DMA,
            pltpu.SemaphoreType.DMA,
        ) + extra,
    )
    def seg_kernel(u_hbm, src_hbm, dst_hbm, *rest):
        if gather_q:
            (q_hbm, g_out, uq_out,
             acc, sslab, dstage, dslab, rowsa, rowsb,
             sga, sgb, ssa, ssb, qidx, qrows) = rest
        else:
            (s_out, acc, sslab, dstage, dslab, rowsa, rowsb,
             sga, sgb, ssa, ssb) = rest

        c = lax.axis_index("c")
        s = lax.axis_index("s")

        base = pl.multiple_of((c * _NS + s) * (_NCH * _CHUNK), 8)
        pltpu.sync_copy(src_hbm.at[pl.ds(base, _NCH * _CHUNK)], sslab)

        def rep_batch(b, carry):
            bo = pl.multiple_of(base + b * _DB, 8)
            pltpu.sync_copy(dst_hbm.at[pl.ds(bo, _DB)], dstage)

            def rep_row(j, carry2):
                for k in range(_CHUNK // 16):
                    dslab[b * _DBR + j, pl.ds(k * 16, 16)] = (
                        dstage[pl.ds(j * _CHUNK + k * 16, 16)])
                return carry2
            lax.fori_loop(0, _DBR, rep_row, 0)
            return carry
        lax.fori_loop(0, _NCH // _DBR, rep_batch, 0)

        def zero_body(r, carry):
            for l in range(_D // 16):
                rowsa[r, pl.ds(l * 16, 16)] = jnp.zeros((16,), _F32)
            return carry
        lax.fori_loop(0, _CHUNK, zero_body, 0)

        def init(r0, nr):
            def cp(k, carry):
                rr = pl.multiple_of(r0 + k * 64, 8)
                pltpu.sync_copy(rowsa.at[pl.ds(0, 64), :],
                                acc.at[pl.ds(rr, 64), :])
                return carry
            lax.fori_loop(0, nr // 64, cp, 0)
        _for_stripe(s, init)
        plsc.subcore_barrier()

        def gather(j, buf, sem):
            jo = pl.multiple_of(j * _CHUNK, 8)
            pltpu.async_copy(u_hbm.at[sslab.at[pl.ds(jo, _CHUNK)]], buf, sem)

        def gwait(buf, sem):
            pltpu.make_async_copy(
                u_hbm.at[sslab.at[pl.ds(0, _CHUNK)]], buf, sem).wait()

        def ascat(j, buf, sem):
            pltpu.async_copy(buf, acc.at[dslab.at[j]], sem, add=True)

        def swait(buf, sem):
            pltpu.make_async_copy(buf, acc.at[dslab.at[0]], sem).wait()

        gather(0, rowsa, sga)
        gather(1, rowsb, sgb)

        def edge_body(g, carry):
            j = 2 * g
            gwait(rowsa, sga)
            ascat(j, rowsa, ssa)
            gwait(rowsb, sgb)
            ascat(j + 1, rowsb, ssb)
            swait(rowsa, ssa)
            gather(j + 2, rowsa, sga)
            swait(rowsb, ssb)

            @pl.when(j + 3 < _NCH)
            def _():
                gather(j + 3, rowsb, sgb)
            return carry
        lax.fori_loop(0, (_NCH - 1) // 2, edge_body, 0)
        gwait(rowsa, sga)
        ascat(_NCH - 1, rowsa, ssa)
        swait(rowsa, ssa)
        plsc.subcore_barrier()

        if gather_q:
            qb = pl.multiple_of(s * _QC, 8)
            for b in range(8):
                qo = pl.multiple_of(qb + b * (_QC // 8), 8)
                pltpu.sync_copy(q_hbm.at[pl.ds(qo, _QC // 8)], qidx.at[b])
                pltpu.async_copy(acc.at[qidx.at[b]], qrows, sga).wait()
                pltpu.sync_copy(qrows, g_out.at[c, pl.ds(qo, _QC // 8), :])

            @pl.when(c == 1)
            def _():
                for b in range(8):
                    qo = pl.multiple_of(qb + b * (_QC // 8), 8)
                    pltpu.async_copy(u_hbm.at[qidx.at[b]], qrows, sga).wait()
                    pltpu.sync_copy(qrows, uq_out.at[pl.ds(qo, _QC // 8), :])
        else:
            def wout(r0, nr):
                def cp(k, carry):
                    rr = pl.multiple_of(r0 + k * 64, 8)
                    pltpu.sync_copy(acc.at[pl.ds(rr, 64), :],
                                    rowsa.at[pl.ds(0, 64), :])
                    pltpu.sync_copy(rowsa.at[pl.ds(0, 64), :],
                                    s_out.at[c, pl.ds(rr, 64), :])
                    return carry
                lax.fori_loop(0, nr // 64, cp, 0)
            _for_stripe(s, wout)

    return seg_kernel


_deg_call = _build_deg()
_seg1_call = _build_seg(gather_q=False)
_seg2_call = _build_seg(gather_q=True)


_BR = 1000


def _tc1(x, w_in, b_in, w1, degp):
    def body(x_ref, win_ref, bin_ref, w1_ref, deg_ref, out_ref):
        h = jnp.maximum(
            jnp.dot(x_ref[...], win_ref[...],
                    preferred_element_type=_F32) + bin_ref[...], 0.0)
        dinv = lax.rsqrt(deg_ref[0] + deg_ref[1] + 1.0)
        out_ref[...] = jnp.dot(h, w1_ref[...],
                               preferred_element_type=_F32) * dinv

    return pl.pallas_call(
        body,
        grid=(_N // _BR,),
        in_specs=[pl.BlockSpec((_BR, _D), lambda i: (i, 0)),
                  pl.BlockSpec((_D, _D), lambda i: (0, 0)),
                  pl.BlockSpec((1, _D), lambda i: (0, 0)),
                  pl.BlockSpec((_D, _D), lambda i: (0, 0)),
                  pl.BlockSpec((_NC, _BR, 1), lambda i: (0, i, 0))],
        out_specs=pl.BlockSpec((_BR, _D), lambda i: (i, 0)),
        out_shape=jax.ShapeDtypeStruct((_N, _D), _F32),
    )(x, w_in, b_in, w1, degp)


def _tc2(sparts, u1, degp, b1, w2):
    def body(sp_ref, u1_ref, deg_ref, b1_ref, w2_ref, out_ref):
        dinv = lax.rsqrt(deg_ref[0] + deg_ref[1] + 1.0)
        st = sp_ref[0] + sp_ref[1] + u1_ref[...]
        h = jnp.maximum(st * dinv + b1_ref[...], 0.0)
        out_ref[...] = jnp.dot(h, w2_ref[...],
                               preferred_element_type=_F32) * dinv

    return pl.pallas_call(
        body,
        grid=(_N // _BR,),
        in_specs=[pl.BlockSpec((_NC, _BR, _D), lambda i: (0, i, 0)),
                  pl.BlockSpec((_BR, _D), lambda i: (i, 0)),
                  pl.BlockSpec((_NC, _BR, 1), lambda i: (0, i, 0)),
                  pl.BlockSpec((1, _D), lambda i: (0, 0)),
                  pl.BlockSpec((_D, _D), lambda i: (0, 0))],
        out_specs=pl.BlockSpec((_BR, _D), lambda i: (i, 0)),
        out_shape=jax.ShapeDtypeStruct((_N, _D), _F32),
    )(sparts, u1, degp, b1, w2)


def _tc3(g, uq, dqp, b2, w_out, b_out):
    def body(g_ref, uq_ref, dq_ref, b2_ref, wout_ref, bout_ref, out_ref):
        dinv = lax.rsqrt(dq_ref[0] + dq_ref[1] + 1.0)
        st = g_ref[0] + g_ref[1] + uq_ref[...]
        h = jnp.maximum(st * dinv + b2_ref[...], 0.0)
        out_ref[...] = jnp.dot(h, wout_ref[...],
                               preferred_element_type=_F32) + bout_ref[...]

    return pl.pallas_call(
        body,
        grid=(1,),
        in_specs=[pl.BlockSpec((_NC, _Q, _D), lambda i: (0, 0, 0)),
                  pl.BlockSpec((_Q, _D), lambda i: (0, 0)),
                  pl.BlockSpec((_NC, _Q, 1), lambda i: (0, 0, 0)),
                  pl.BlockSpec((1, _D), lambda i: (0, 0)),
                  pl.BlockSpec((_D, _D), lambda i: (0, 0)),
                  pl.BlockSpec((1, _D), lambda i: (0, 0))],
        out_specs=pl.BlockSpec((_Q, _D), lambda i: (0, 0)),
        out_shape=jax.ShapeDtypeStruct((_Q, _D), _F32),
    )(g, uq, dqp, b2, w_out, b_out)


def kernel(x, current_node_idx, edge_index, W_in, b_in, W1, b1, W2, b2,
           W_out, b_out):
    src1 = edge_index[0].astype(jnp.int32)
    dst1 = edge_index[1].astype(jnp.int32)
    q = current_node_idx.astype(jnp.int32)

    degp, dqp = _deg_call(dst1, q)
    degp3 = degp.reshape(_NC, _N, 1)

    u1 = _tc1(x, W_in, b_in.reshape(1, _D), W1, degp3)
    sparts = _seg1_call(u1, src1, dst1)
    u2 = _tc2(sparts, u1, degp3, b1.reshape(1, _D), W2)
    g, uq = _seg2_call(u2, src1, dst1, q)
    return _tc3(g, uq, dqp.reshape(_NC, _Q, 1), b2.reshape(1, _D),
                W_out, b_out.reshape(1, _D))


# speedup vs baseline: 25.6308x; 1.0426x over previous
import functools

import jax
import jax.numpy as jnp
from jax import lax
from jax.experimental import pallas as pl
from jax.experimental.pallas import tpu as pltpu
from jax.experimental.pallas import tpu_sc as plsc

_N = 10000
_E = 320000
_D = 128
_H = _D // 2
_Q = 1024

_NC = 2
_NS = 16
_CHUNK = 80
_QC = _Q // _NS

_STRIPE = 640
_LAST_OFF = _N - 512
_LAST = 512

_F32 = jnp.float32


def _mesh():
    return plsc.VectorSubcoreMesh(
        core_axis_name="c", subcore_axis_name="s",
        num_cores=_NC, num_subcores=_NS)


def _for_stripe(s, emit):
    @pl.when(s < _NS - 1)
    def _():
        emit(pl.multiple_of(s * _STRIPE, 8), _STRIPE)

    @pl.when(s == _NS - 1)
    def _():
        emit(_LAST_OFF, _LAST)


def _build_deg():
    ndch = _E // _NC // _NS // _CHUNK

    @functools.partial(
        pl.kernel,
        out_type=(jax.ShapeDtypeStruct((_NC * _N,), _F32),
                  jax.ShapeDtypeStruct((_NC * _Q,), _F32)),
        mesh=_mesh(),
        scratch_types=(
            pltpu.VMEM_SHARED((_N,), _F32),
            pltpu.VMEM((_STRIPE,), _F32),
            pltpu.VMEM((ndch * _CHUNK,), jnp.int32),
            pltpu.VMEM((ndch, _CHUNK), jnp.int32),
            pltpu.VMEM((_CHUNK,), _F32),
            pltpu.VMEM((1, _QC), jnp.int32),
            pltpu.VMEM((_QC,), _F32),
            pltpu.SemaphoreType.DMA,
        ),
    )
    def deg_kernel(dst_hbm, q_hbm, deg_out, dq_out,
                   deg_sp, stage, dslab1, dslab, ones, qidx, dqv, sem):
        c = lax.axis_index("c")
        s = lax.axis_index("s")

        def zero_body(k, carry):
            stage[pl.ds(k * 16, 16)] = jnp.zeros((16,), _F32)
            return carry
        lax.fori_loop(0, _STRIPE // 16, zero_body, 0)
        for k in range(_CHUNK // 16):
            ones[pl.ds(k * 16, 16)] = jnp.full((16,), 1.0, _F32)

        base = pl.multiple_of((c * _NS + s) * (ndch * _CHUNK), 8)
        pltpu.sync_copy(dst_hbm.at[pl.ds(base, ndch * _CHUNK)], dslab1)

        def repack(j, carry):
            for k in range(_CHUNK // 16):
                dslab[j, pl.ds(k * 16, 16)] = (
                    dslab1[pl.ds(j * _CHUNK + k * 16, 16)])
            return carry
        lax.fori_loop(0, ndch, repack, 0)

        def init(r0, nr):
            pltpu.sync_copy(stage.at[pl.ds(0, nr)], deg_sp.at[pl.ds(r0, nr)])
        _for_stripe(s, init)
        plsc.subcore_barrier()

        def edge_body(j, carry):
            pltpu.sync_copy(ones, deg_sp.at[dslab.at[j]], add=True)
            return carry
        lax.fori_loop(0, ndch, edge_body, 0)
        plsc.subcore_barrier()

        def wout(r0, nr):
            pltpu.sync_copy(deg_sp.at[pl.ds(r0, nr)], stage.at[pl.ds(0, nr)])
            o0 = pl.multiple_of(c * _N + r0, 8)
            pltpu.sync_copy(stage.at[pl.ds(0, nr)], deg_out.at[pl.ds(o0, nr)])
        _for_stripe(s, wout)
        qb = pl.multiple_of(s * _QC, 8)
        pltpu.sync_copy(q_hbm.at[pl.ds(qb, _QC)], qidx.at[0])
        pltpu.async_copy(deg_sp.at[qidx.at[0]], dqv, sem).wait()
        oq = pl.multiple_of(c * _Q + qb, 8)
        pltpu.sync_copy(dqv, dq_out.at[pl.ds(oq, _QC)])

    return deg_kernel


_NCH = _E // _NS // _CHUNK
_DB = 2000
_DBR = _DB // _CHUNK
_QB = 16


def _build_seg(gather_q):
    if gather_q:
        out_type = (jax.ShapeDtypeStruct((_NC, _Q, _H), _F32),
                    jax.ShapeDtypeStruct((_NC, _Q, _H), _F32))
        extra = (pltpu.VMEM((_QC // _QB, _QB), jnp.int32),
                 pltpu.VMEM((_QC // _QB, _QB), jnp.int32),
                 pltpu.VMEM((_QB, _H), _F32))
    else:
        out_type = jax.ShapeDtypeStruct((_NC, _N, _H), _F32)
        extra = ()

    @functools.partial(
        pl.kernel,
        out_type=out_type,
        mesh=_mesh(),
        compiler_params=pltpu.CompilerParams(use_tc_tiling_on_sc=False),
        scratch_types=(
            pltpu.VMEM_SHARED((_N, _H), _F32),
            pltpu.VMEM((_DB,), jnp.int32),
            pltpu.VMEM((_NCH, _CHUNK), jnp.int32),
            pltpu.VMEM((_NCH, _CHUNK), jnp.int32),
            pltpu.VMEM((_CHUNK, _H), _F32),
            pltpu.VMEM((_CHUNK, _H), _F32),
            pltpu.VMEM((_CHUNK, _H), _F32),
            pltpu.VMEM((_CHUNK, _H), _F32),
            pltpu.SemaphoreType.DMA,
            pltpu.SemaphoreType.DMA,
            pltpu.SemaphoreType.DMA,
            pltpu.SemaphoreType.DMA,
            pltpu.SemaphoreType.DMA,
            pltpu.SemaphoreType.DMA,
            pltpu.SemaphoreType.DMA,
            pltpu.SemaphoreType.DMA,
        ) + extra,
    )
    def seg_kernel(u_hbm, src_hbm, dst_hbm, *rest):
        if gather_q:
            (q_hbm, g_out, uq_out, acc, stage, sslab, dslab,
             b0, b1, b2, b3, g0, g1, g2, g3, s0, s1, s2, s3,
             qidx, qidx2, qrows) = rest
        else:
            (s_out, acc, stage, sslab, dslab,
             b0, b1, b2, b3, g0, g1, g2, g3, s0, s1, s2, s3) = rest
        bufs = (b0, b1, b2, b3)
        gsem = (g0, g1, g2, g3)
        ssem = (s0, s1, s2, s3)

        c = lax.axis_index("c")
        s = lax.axis_index("s")
        coff = c * _N

        base = pl.multiple_of(s * (_NCH * _CHUNK), 8)

        def load_slab(hbm, slab, off):
            def rep_batch(b, carry):
                bo = pl.multiple_of(base + b * _DB, 8)
                pltpu.sync_copy(hbm.at[pl.ds(bo, _DB)], stage)

                def rep_row(j, carry2):
                    for k in range(_CHUNK // 16):
                        slab[b * _DBR + j, pl.ds(k * 16, 16)] = (
                            stage[pl.ds(j * _CHUNK + k * 16, 16)] + off)
                    return carry2
                lax.fori_loop(0, _DBR, rep_row, 0)
                return carry
            lax.fori_loop(0, _NCH // _DBR, rep_batch, 0)

        load_slab(src_hbm, sslab, coff)
        load_slab(dst_hbm, dslab, 0)

        def zero_body(r, carry):
            for l in range(_H // 16):
                b0[r, pl.ds(l * 16, 16)] = jnp.zeros((16,), _F32)
            return carry
        lax.fori_loop(0, _CHUNK, zero_body, 0)

        def init(r0, nr):
            def cp(k, carry):
                rr = pl.multiple_of(r0 + k * 64, 8)
                pltpu.sync_copy(b0.at[pl.ds(0, 64), :],
                                acc.at[pl.ds(rr, 64), :])
                return carry
            lax.fori_loop(0, nr // 64, cp, 0)
        _for_stripe(s, init)
        plsc.subcore_barrier()

        def gather(j, buf, sem):
            pltpu.async_copy(u_hbm.at[sslab.at[j]], buf, sem)

        def gwait(buf, sem):
            pltpu.make_async_copy(u_hbm.at[sslab.at[0]], buf, sem).wait()

        def ascat(j, buf, sem):
            pltpu.async_copy(buf, acc.at[dslab.at[j]], sem, add=True)

        def swait(buf, sem):
            pltpu.make_async_copy(buf, acc.at[dslab.at[0]], sem).wait()

        for t in range(4):
            gather(t, bufs[t], gsem[t])

        def edge_body(g, carry):
            j = 4 * g
            for t in range(4):
                gwait(bufs[t], gsem[t])
                ascat(j + t, bufs[t], ssem[t])
            for t in range(4):
                swait(bufs[t], ssem[t])

                @pl.when(j + 4 + t < _NCH)
                def _():
                    gather(j + 4 + t, bufs[t], gsem[t])
            return carry
        lax.fori_loop(0, _NCH // 4, edge_body, 0)
        for t in range(2):
            gwait(bufs[t], gsem[t])
            ascat(_NCH - 2 + t, bufs[t], ssem[t])
        for t in range(2):
            swait(bufs[t], ssem[t])
        plsc.subcore_barrier()

        if gather_q:
            qb = pl.multiple_of(s * _QC, 8)
            for b in range(_QC // _QB):
                qo = pl.multiple_of(qb + b * _QB, 8)
                pltpu.sync_copy(q_hbm.at[pl.ds(qo, _QB)], qidx.at[b])

            def adj(b, carry):
                qidx2[b, pl.ds(0, 16)] = qidx[b, pl.ds(0, 16)] + coff
                return carry
            lax.fori_loop(0, _QC // _QB, adj, 0)
            for b in range(_QC // _QB):
                qo = pl.multiple_of(qb + b * _QB, 8)
                pltpu.async_copy(acc.at[qidx.at[b]], qrows, g0).wait()
                pltpu.sync_copy(qrows, g_out.at[c, pl.ds(qo, _QB), :])
                pltpu.async_copy(u_hbm.at[qidx2.at[b]], qrows, g0).wait()
                pltpu.sync_copy(qrows, uq_out.at[c, pl.ds(qo, _QB), :])
        else:
            def wout(r0, nr):
                def cp(k, carry):
                    rr = pl.multiple_of(r0 + k * 64, 8)
                    pltpu.sync_copy(acc.at[pl.ds(rr, 64), :],
                                    b1.at[pl.ds(0, 64), :])
                    pltpu.sync_copy(b1.at[pl.ds(0, 64), :],
                                    s_out.at[c, pl.ds(rr, 64), :])
                    return carry
                lax.fori_loop(0, nr // 64, cp, 0)
            _for_stripe(s, wout)

    return seg_kernel


_deg_call = _build_deg()
_seg1_call = _build_seg(gather_q=False)
_seg2_call = _build_seg(gather_q=True)


_BR = 1000


def _tc1(x, w_in, b_in, w1, degp):
    def body(x_ref, win_ref, bin_ref, w1_ref, deg_ref, out_ref):
        h = jnp.maximum(
            jnp.dot(x_ref[...], win_ref[...],
                    preferred_element_type=_F32) + bin_ref[...], 0.0)
        dinv = lax.rsqrt(deg_ref[0] + deg_ref[1] + 1.0)
        u = jnp.dot(h, w1_ref[...], preferred_element_type=_F32) * dinv
        out_ref[0] = u[:, :_H]
        out_ref[1] = u[:, _H:]

    return pl.pallas_call(
        body,
        grid=(_N // _BR,),
        in_specs=[pl.BlockSpec((_BR, _D), lambda i: (i, 0)),
                  pl.BlockSpec((_D, _D), lambda i: (0, 0)),
                  pl.BlockSpec((1, _D), lambda i: (0, 0)),
                  pl.BlockSpec((_D, _D), lambda i: (0, 0)),
                  pl.BlockSpec((_NC, _BR, 1), lambda i: (0, i, 0))],
        out_specs=pl.BlockSpec((_NC, _BR, _H), lambda i: (0, i, 0)),
        out_shape=jax.ShapeDtypeStruct((_NC, _N, _H), _F32),
    )(x, w_in, b_in, w1, degp)


def _tc2(sparts, u1, degp, b1, w2):
    def body(sp_ref, u1_ref, deg_ref, b1_ref, w2_ref, out_ref):
        dinv = lax.rsqrt(deg_ref[0] + deg_ref[1] + 1.0)
        st = jnp.concatenate(
            [sp_ref[0] + u1_ref[0], sp_ref[1] + u1_ref[1]], axis=1)
        h = jnp.maximum(st * dinv + b1_ref[...], 0.0)
        u = jnp.dot(h, w2_ref[...], preferred_element_type=_F32) * dinv
        out_ref[0] = u[:, :_H]
        out_ref[1] = u[:, _H:]

    return pl.pallas_call(
        body,
        grid=(_N // _BR,),
        in_specs=[pl.BlockSpec((_NC, _BR, _H), lambda i: (0, i, 0)),
                  pl.BlockSpec((_NC, _BR, _H), lambda i: (0, i, 0)),
                  pl.BlockSpec((_NC, _BR, 1), lambda i: (0, i, 0)),
                  pl.BlockSpec((1, _D), lambda i: (0, 0)),
                  pl.BlockSpec((_D, _D), lambda i: (0, 0))],
        out_specs=pl.BlockSpec((_NC, _BR, _H), lambda i: (0, i, 0)),
        out_shape=jax.ShapeDtypeStruct((_NC, _N, _H), _F32),
    )(sparts, u1, degp, b1, w2)


def _tc3(g, uq, dqp, b2, w_out, b_out):
    def body(g_ref, uq_ref, dq_ref, b2_ref, wout_ref, bout_ref, out_ref):
        dinv = lax.rsqrt(dq_ref[0] + dq_ref[1] + 1.0)
        st = jnp.concatenate(
            [g_ref[0] + uq_ref[0], g_ref[1] + uq_ref[1]], axis=1)
        h = jnp.maximum(st * dinv + b2_ref[...], 0.0)
        out_ref[...] = jnp.dot(h, wout_ref[...],
                               preferred_element_type=_F32) + bout_ref[...]

    return pl.pallas_call(
        body,
        grid=(1,),
        in_specs=[pl.BlockSpec((_NC, _Q, _H), lambda i: (0, 0, 0)),
                  pl.BlockSpec((_NC, _Q, _H), lambda i: (0, 0, 0)),
                  pl.BlockSpec((_NC, _Q, 1), lambda i: (0, 0, 0)),
                  pl.BlockSpec((1, _D), lambda i: (0, 0)),
                  pl.BlockSpec((_D, _D), lambda i: (0, 0)),
                  pl.BlockSpec((1, _D), lambda i: (0, 0))],
        out_specs=pl.BlockSpec((_Q, _D), lambda i: (0, 0)),
        out_shape=jax.ShapeDtypeStruct((_Q, _D), _F32),
    )(g, uq, dqp, b2, w_out, b_out)


def kernel(x, current_node_idx, edge_index, W_in, b_in, W1, b1, W2, b2,
           W_out, b_out):
    src1 = edge_index[0].astype(jnp.int32)
    dst1 = edge_index[1].astype(jnp.int32)
    q = current_node_idx.astype(jnp.int32)

    degp, dqp = _deg_call(dst1, q)
    degp3 = degp.reshape(_NC, _N, 1)

    u1 = _tc1(x, W_in, b_in.reshape(1, _D), W1, degp3)
    sparts = _seg1_call(u1.reshape(_NC * _N, _H), src1, dst1)
    u2 = _tc2(sparts, u1, degp3, b1.reshape(1, _D), W2)
    g, uq = _seg2_call(u2.reshape(_NC * _N, _H), src1, dst1, q)
    return _tc3(g, uq, dqp.reshape(_NC, _Q, 1), b2.reshape(1, _D),
                W_out, b_out.reshape(1, _D))


# speedup vs baseline: 28.3129x; 1.1046x over previous
import functools

import jax
import jax.numpy as jnp
from jax import lax
from jax.experimental import pallas as pl
from jax.experimental.pallas import tpu as pltpu
from jax.experimental.pallas import tpu_sc as plsc

_N = 10000
_E = 320000
_D = 128
_H = _D // 2
_Q = 1024

_NC = 2
_NS = 16
_CHUNK = 80
_QC = _Q // _NS

_STRIPE = 640
_LAST_OFF = _N - 512
_LAST = 512

_F32 = jnp.float32


def _mesh():
    return plsc.VectorSubcoreMesh(
        core_axis_name="c", subcore_axis_name="s",
        num_cores=_NC, num_subcores=_NS)


def _for_stripe(s, emit):
    @pl.when(s < _NS - 1)
    def _():
        emit(pl.multiple_of(s * _STRIPE, 8), _STRIPE)

    @pl.when(s == _NS - 1)
    def _():
        emit(_LAST_OFF, _LAST)


def _build_deg():
    ndch = _E // _NC // _NS // _CHUNK

    @functools.partial(
        pl.kernel,
        out_type=(jax.ShapeDtypeStruct((_NC * _N,), _F32),
                  jax.ShapeDtypeStruct((_NC * _Q,), _F32)),
        mesh=_mesh(),
        scratch_types=(
            pltpu.VMEM_SHARED((_N,), _F32),
            pltpu.VMEM((_STRIPE,), _F32),
            pltpu.VMEM((ndch * _CHUNK,), jnp.int32),
            pltpu.VMEM((ndch, _CHUNK), jnp.int32),
            pltpu.VMEM((_CHUNK,), _F32),
            pltpu.VMEM((1, _QC), jnp.int32),
            pltpu.VMEM((_QC,), _F32),
            pltpu.SemaphoreType.DMA,
        ),
    )
    def deg_kernel(dst_hbm, q_hbm, deg_out, dq_out,
                   deg_sp, stage, dslab1, dslab, ones, qidx, dqv, sem):
        c = lax.axis_index("c")
        s = lax.axis_index("s")

        def zero_body(k, carry):
            stage[pl.ds(k * 16, 16)] = jnp.zeros((16,), _F32)
            return carry
        lax.fori_loop(0, _STRIPE // 16, zero_body, 0)
        for k in range(_CHUNK // 16):
            ones[pl.ds(k * 16, 16)] = jnp.full((16,), 1.0, _F32)

        base = pl.multiple_of((c * _NS + s) * (ndch * _CHUNK), 8)
        pltpu.sync_copy(dst_hbm.at[pl.ds(base, ndch * _CHUNK)], dslab1)

        def repack(j, carry):
            for k in range(_CHUNK // 16):
                dslab[j, pl.ds(k * 16, 16)] = (
                    dslab1[pl.ds(j * _CHUNK + k * 16, 16)])
            return carry
        lax.fori_loop(0, ndch, repack, 0)

        def init(r0, nr):
            pltpu.sync_copy(stage.at[pl.ds(0, nr)], deg_sp.at[pl.ds(r0, nr)])
        _for_stripe(s, init)
        plsc.subcore_barrier()

        def edge_body(j, carry):
            pltpu.sync_copy(ones, deg_sp.at[dslab.at[j]], add=True)
            return carry
        lax.fori_loop(0, ndch, edge_body, 0)
        plsc.subcore_barrier()

        def wout(r0, nr):
            pltpu.sync_copy(deg_sp.at[pl.ds(r0, nr)], stage.at[pl.ds(0, nr)])
            o0 = pl.multiple_of(c * _N + r0, 8)
            pltpu.sync_copy(stage.at[pl.ds(0, nr)], deg_out.at[pl.ds(o0, nr)])
        _for_stripe(s, wout)
        qb = pl.multiple_of(s * _QC, 8)
        pltpu.sync_copy(q_hbm.at[pl.ds(qb, _QC)], qidx.at[0])
        pltpu.async_copy(deg_sp.at[qidx.at[0]], dqv, sem).wait()
        oq = pl.multiple_of(c * _Q + qb, 8)
        pltpu.sync_copy(dqv, dq_out.at[pl.ds(oq, _QC)])

    return deg_kernel


_NCH = _E // _NS // _CHUNK
_DB = 2000
_DBR = _DB // _CHUNK
_QB = 16


def _build_seg(gather_q):
    if gather_q:
        out_type = (jax.ShapeDtypeStruct((_Q, _D), _F32),
                    jax.ShapeDtypeStruct((_Q, _D), _F32))
        extra = (pltpu.VMEM((_QC // _QB, _QB), jnp.int32),
                 pltpu.VMEM((_QC // _QB, _QB), jnp.int32),
                 pltpu.VMEM((_QB, _H), _F32))
    else:
        out_type = jax.ShapeDtypeStruct((_N, _D), _F32)
        extra = ()

    @functools.partial(
        pl.kernel,
        out_type=out_type,
        mesh=_mesh(),
        compiler_params=pltpu.CompilerParams(use_tc_tiling_on_sc=False),
        scratch_types=(
            pltpu.VMEM_SHARED((_N, _H), _F32),
            pltpu.VMEM((_DB,), jnp.int32),
            pltpu.VMEM((_NCH, _CHUNK), jnp.int32),
            pltpu.VMEM((_NCH, _CHUNK), jnp.int32),
            pltpu.VMEM((_CHUNK, _H), _F32),
            pltpu.VMEM((_CHUNK, _H), _F32),
            pltpu.VMEM((_CHUNK, _H), _F32),
            pltpu.VMEM((_CHUNK, _H), _F32),
            pltpu.SemaphoreType.DMA,
            pltpu.SemaphoreType.DMA,
            pltpu.SemaphoreType.DMA,
            pltpu.SemaphoreType.DMA,
            pltpu.SemaphoreType.DMA,
            pltpu.SemaphoreType.DMA,
            pltpu.SemaphoreType.DMA,
            pltpu.SemaphoreType.DMA,
        ) + extra,
    )
    def seg_kernel(u_hbm, src_hbm, dst_hbm, *rest):
        if gather_q:
            (q_hbm, g_out, uq_out, acc, stage, sslab, dslab,
             b0, b1, b2, b3, g0, g1, g2, g3, s0, s1, s2, s3,
             qidx, qidx2, qrows) = rest
        else:
            (s_out, acc, stage, sslab, dslab,
             b0, b1, b2, b3, g0, g1, g2, g3, s0, s1, s2, s3) = rest
        bufs = (b0, b1, b2, b3)
        gsem = (g0, g1, g2, g3)
        ssem = (s0, s1, s2, s3)

        c = lax.axis_index("c")
        s = lax.axis_index("s")
        ccol = pl.multiple_of(c * _H, 8)

        base = pl.multiple_of(s * (_NCH * _CHUNK), 8)

        def load_slab(hbm, slab, mul, off):
            def rep_batch(b, carry):
                bo = pl.multiple_of(base + b * _DB, 8)
                pltpu.sync_copy(hbm.at[pl.ds(bo, _DB)], stage)

                def rep_row(j, carry2):
                    for k in range(_CHUNK // 16):
                        slab[b * _DBR + j, pl.ds(k * 16, 16)] = (
                            stage[pl.ds(j * _CHUNK + k * 16, 16)] * mul
                            + off)
                    return carry2
                lax.fori_loop(0, _DBR, rep_row, 0)
                return carry
            lax.fori_loop(0, _NCH // _DBR, rep_batch, 0)

        load_slab(src_hbm, sslab, 2, c)
        load_slab(dst_hbm, dslab, 1, 0)

        def zero_body(r, carry):
            for l in range(_H // 16):
                b0[r, pl.ds(l * 16, 16)] = jnp.zeros((16,), _F32)
            return carry
        lax.fori_loop(0, _CHUNK, zero_body, 0)

        def init(r0, nr):
            def cp(k, carry):
                rr = pl.multiple_of(r0 + k * 64, 8)
                pltpu.sync_copy(b0.at[pl.ds(0, 64), :],
                                acc.at[pl.ds(rr, 64), :])
                return carry
            lax.fori_loop(0, nr // 64, cp, 0)
        _for_stripe(s, init)
        plsc.subcore_barrier()

        def gather(j, buf, sem):
            pltpu.async_copy(u_hbm.at[sslab.at[j]], buf, sem)

        def gwait(buf, sem):
            pltpu.make_async_copy(u_hbm.at[sslab.at[0]], buf, sem).wait()

        def ascat(j, buf, sem):
            pltpu.async_copy(buf, acc.at[dslab.at[j]], sem, add=True)

        def swait(buf, sem):
            pltpu.make_async_copy(buf, acc.at[dslab.at[0]], sem).wait()

        for t in range(4):
            gather(t, bufs[t], gsem[t])

        def edge_body(g, carry):
            j = 4 * g
            for t in range(4):
                gwait(bufs[t], gsem[t])
                ascat(j + t, bufs[t], ssem[t])
            for t in range(4):
                swait(bufs[t], ssem[t])

                @pl.when(j + 4 + t < _NCH)
                def _():
                    gather(j + 4 + t, bufs[t], gsem[t])
            return carry
        lax.fori_loop(0, _NCH // 4, edge_body, 0)
        for t in range(2):
            gwait(bufs[t], gsem[t])
            ascat(_NCH - 2 + t, bufs[t], ssem[t])
        for t in range(2):
            swait(bufs[t], ssem[t])
        plsc.subcore_barrier()

        if gather_q:
            qb = pl.multiple_of(s * _QC, 8)
            for b in range(_QC // _QB):
                qo = pl.multiple_of(qb + b * _QB, 8)
                pltpu.sync_copy(q_hbm.at[pl.ds(qo, _QB)], qidx.at[b])

            def adj(b, carry):
                qidx2[b, pl.ds(0, 16)] = qidx[b, pl.ds(0, 16)] * 2 + c
                return carry
            lax.fori_loop(0, _QC // _QB, adj, 0)
            for b in range(_QC // _QB):
                qo = pl.multiple_of(qb + b * _QB, 8)
                pltpu.async_copy(acc.at[qidx.at[b]], qrows, g0).wait()
                pltpu.sync_copy(
                    qrows, g_out.at[pl.ds(qo, _QB), pl.ds(ccol, _H)])
                pltpu.async_copy(u_hbm.at[qidx2.at[b]], qrows, g0).wait()
                pltpu.sync_copy(
                    qrows, uq_out.at[pl.ds(qo, _QB), pl.ds(ccol, _H)])
        else:
            def wout(r0, nr):
                def cp(k, carry):
                    rr = pl.multiple_of(r0 + k * 64, 8)
                    pltpu.sync_copy(acc.at[pl.ds(rr, 64), :],
                                    b1.at[pl.ds(0, 64), :])
                    pltpu.sync_copy(
                        b1.at[pl.ds(0, 64), :],
                        s_out.at[pl.ds(rr, 64), pl.ds(ccol, _H)])
                    return carry
                lax.fori_loop(0, nr // 64, cp, 0)
            _for_stripe(s, wout)

    return seg_kernel


_deg_call = _build_deg()
_seg1_call = _build_seg(gather_q=False)
_seg2_call = _build_seg(gather_q=True)


_BR = 1000


def _tc1(x, w_in, b_in, w1, degp):
    def body(x_ref, win_ref, bin_ref, w1_ref, deg_ref, out_ref):
        h = jnp.maximum(
            jnp.dot(x_ref[...], win_ref[...],
                    preferred_element_type=_F32) + bin_ref[...], 0.0)
        dinv = lax.rsqrt(deg_ref[0] + deg_ref[1] + 1.0)
        out_ref[...] = jnp.dot(h, w1_ref[...],
                               preferred_element_type=_F32) * dinv

    return pl.pallas_call(
        body,
        grid=(_N // _BR,),
        in_specs=[pl.BlockSpec((_BR, _D), lambda i: (i, 0)),
                  pl.BlockSpec((_D, _D), lambda i: (0, 0)),
                  pl.BlockSpec((1, _D), lambda i: (0, 0)),
                  pl.BlockSpec((_D, _D), lambda i: (0, 0)),
                  pl.BlockSpec((_NC, _BR, 1), lambda i: (0, i, 0))],
        out_specs=pl.BlockSpec((_BR, _D), lambda i: (i, 0)),
        out_shape=jax.ShapeDtypeStruct((_N, _D), _F32),
    )(x, w_in, b_in, w1, degp)


def _tc2(sfull, u1, degp, b1, w2):
    def body(s_ref, u1_ref, deg_ref, b1_ref, w2_ref, out_ref):
        dinv = lax.rsqrt(deg_ref[0] + deg_ref[1] + 1.0)
        st = s_ref[...] + u1_ref[...]
        h = jnp.maximum(st * dinv + b1_ref[...], 0.0)
        out_ref[...] = jnp.dot(h, w2_ref[...],
                               preferred_element_type=_F32) * dinv

    return pl.pallas_call(
        body,
        grid=(_N // _BR,),
        in_specs=[pl.BlockSpec((_BR, _D), lambda i: (i, 0)),
                  pl.BlockSpec((_BR, _D), lambda i: (i, 0)),
                  pl.BlockSpec((_NC, _BR, 1), lambda i: (0, i, 0)),
                  pl.BlockSpec((1, _D), lambda i: (0, 0)),
                  pl.BlockSpec((_D, _D), lambda i: (0, 0))],
        out_specs=pl.BlockSpec((_BR, _D), lambda i: (i, 0)),
        out_shape=jax.ShapeDtypeStruct((_N, _D), _F32),
    )(sfull, u1, degp, b1, w2)


def _tc3(g, uq, dqp, b2, w_out, b_out):
    def body(g_ref, uq_ref, dq_ref, b2_ref, wout_ref, bout_ref, out_ref):
        dinv = lax.rsqrt(dq_ref[0] + dq_ref[1] + 1.0)
        st = g_ref[...] + uq_ref[...]
        h = jnp.maximum(st * dinv + b2_ref[...], 0.0)
        out_ref[...] = jnp.dot(h, wout_ref[...],
                               preferred_element_type=_F32) + bout_ref[...]

    return pl.pallas_call(
        body,
        grid=(1,),
        in_specs=[pl.BlockSpec((_Q, _D), lambda i: (0, 0)),
                  pl.BlockSpec((_Q, _D), lambda i: (0, 0)),
                  pl.BlockSpec((_NC, _Q, 1), lambda i: (0, 0, 0)),
                  pl.BlockSpec((1, _D), lambda i: (0, 0)),
                  pl.BlockSpec((_D, _D), lambda i: (0, 0)),
                  pl.BlockSpec((1, _D), lambda i: (0, 0))],
        out_specs=pl.BlockSpec((_Q, _D), lambda i: (0, 0)),
        out_shape=jax.ShapeDtypeStruct((_Q, _D), _F32),
    )(g, uq, dqp, b2, w_out, b_out)


def kernel(x, current_node_idx, edge_index, W_in, b_in, W1, b1, W2, b2,
           W_out, b_out):
    src1 = edge_index[0].astype(jnp.int32)
    dst1 = edge_index[1].astype(jnp.int32)
    q = current_node_idx.astype(jnp.int32)

    degp, dqp = _deg_call(dst1, q)
    degp3 = degp.reshape(_NC, _N, 1)

    u1 = _tc1(x, W_in, b_in.reshape(1, _D), W1, degp3)
    sfull = _seg1_call(u1.reshape(2 * _N, _H), src1, dst1)
    u2 = _tc2(sfull, u1, degp3, b1.reshape(1, _D), W2)
    g, uq = _seg2_call(u2.reshape(2 * _N, _H), src1, dst1, q)
    return _tc3(g, uq, dqp.reshape(_NC, _Q, 1), b2.reshape(1, _D),
                W_out, b_out.reshape(1, _D))


# speedup vs baseline: 28.5001x; 1.0066x over previous
import functools

import jax
import jax.numpy as jnp
from jax import lax
from jax.experimental import pallas as pl
from jax.experimental.pallas import tpu as pltpu
from jax.experimental.pallas import tpu_sc as plsc

_N = 10000
_E = 320000
_D = 128
_H = _D // 2
_Q = 1024

_NC = 2
_NS = 16
_CHUNK = 80
_QC = _Q // _NS

_STRIPE = 640
_LAST_OFF = _N - 512
_LAST = 512

_F32 = jnp.float32


def _mesh():
    return plsc.VectorSubcoreMesh(
        core_axis_name="c", subcore_axis_name="s",
        num_cores=_NC, num_subcores=_NS)


def _for_stripe(s, emit):
    @pl.when(s < _NS - 1)
    def _():
        emit(pl.multiple_of(s * _STRIPE, 8), _STRIPE)

    @pl.when(s == _NS - 1)
    def _():
        emit(_LAST_OFF, _LAST)


def _build_deg():
    ndch = _E // _NC // _NS // _CHUNK

    @functools.partial(
        pl.kernel,
        out_type=(jax.ShapeDtypeStruct((_NC * _N,), _F32),
                  jax.ShapeDtypeStruct((_NC * _Q,), _F32)),
        mesh=_mesh(),
        scratch_types=(
            pltpu.VMEM_SHARED((_N,), _F32),
            pltpu.VMEM((_STRIPE,), _F32),
            pltpu.VMEM((ndch * _CHUNK,), jnp.int32),
            pltpu.VMEM((ndch, _CHUNK), jnp.int32),
            pltpu.VMEM((_CHUNK,), _F32),
            pltpu.VMEM((1, _QC), jnp.int32),
            pltpu.VMEM((_QC,), _F32),
            pltpu.SemaphoreType.DMA,
        ),
    )
    def deg_kernel(dst_hbm, q_hbm, deg_out, dq_out,
                   deg_sp, stage, dslab1, dslab, ones, qidx, dqv, sem):
        c = lax.axis_index("c")
        s = lax.axis_index("s")

        def zero_body(k, carry):
            stage[pl.ds(k * 16, 16)] = jnp.zeros((16,), _F32)
            return carry
        lax.fori_loop(0, _STRIPE // 16, zero_body, 0)
        for k in range(_CHUNK // 16):
            ones[pl.ds(k * 16, 16)] = jnp.full((16,), 1.0, _F32)

        base = pl.multiple_of((c * _NS + s) * (ndch * _CHUNK), 8)
        pltpu.sync_copy(dst_hbm.at[pl.ds(base, ndch * _CHUNK)], dslab1)

        def repack(j, carry):
            for k in range(_CHUNK // 16):
                dslab[j, pl.ds(k * 16, 16)] = (
                    dslab1[pl.ds(j * _CHUNK + k * 16, 16)])
            return carry
        lax.fori_loop(0, ndch, repack, 0)

        def init(r0, nr):
            pltpu.sync_copy(stage.at[pl.ds(0, nr)], deg_sp.at[pl.ds(r0, nr)])
        _for_stripe(s, init)
        plsc.subcore_barrier()

        def edge_body(j, carry):
            pltpu.sync_copy(ones, deg_sp.at[dslab.at[j]], add=True)
            return carry
        lax.fori_loop(0, ndch, edge_body, 0)
        plsc.subcore_barrier()

        def wout(r0, nr):
            pltpu.sync_copy(deg_sp.at[pl.ds(r0, nr)], stage.at[pl.ds(0, nr)])
            o0 = pl.multiple_of(c * _N + r0, 8)
            pltpu.sync_copy(stage.at[pl.ds(0, nr)], deg_out.at[pl.ds(o0, nr)])
        _for_stripe(s, wout)
        qb = pl.multiple_of(s * _QC, 8)
        pltpu.sync_copy(q_hbm.at[pl.ds(qb, _QC)], qidx.at[0])
        pltpu.async_copy(deg_sp.at[qidx.at[0]], dqv, sem).wait()
        oq = pl.multiple_of(c * _Q + qb, 8)
        pltpu.sync_copy(dqv, dq_out.at[pl.ds(oq, _QC)])

    return deg_kernel


_NCH = _E // _NS // _CHUNK
_DB = 2000
_DBR = _DB // _CHUNK
_QB = 16


def _build_seg(gather_q):
    if gather_q:
        out_type = (jax.ShapeDtypeStruct((_Q, _D), _F32),
                    jax.ShapeDtypeStruct((_Q, _D), _F32))
        extra = (pltpu.VMEM((_QC // _QB, _QB), jnp.int32),
                 pltpu.VMEM((_QC // _QB, _QB), jnp.int32),
                 pltpu.VMEM((_QB, _H), _F32))
    else:
        out_type = jax.ShapeDtypeStruct((_N, _D), _F32)
        extra = ()

    @functools.partial(
        pl.kernel,
        out_type=out_type,
        mesh=_mesh(),
        compiler_params=pltpu.CompilerParams(use_tc_tiling_on_sc=False),
        scratch_types=(
            pltpu.VMEM_SHARED((_N, _H), _F32),
            pltpu.VMEM((_DB,), jnp.int32),
            pltpu.VMEM((_NCH, _CHUNK), jnp.int32),
            pltpu.VMEM((_NCH, _CHUNK), jnp.int32),
            pltpu.VMEM((_CHUNK, _H), _F32),
            pltpu.VMEM((_CHUNK, _H), _F32),
            pltpu.VMEM((_CHUNK, _H), _F32),
            pltpu.VMEM((_CHUNK, _H), _F32),
            pltpu.SemaphoreType.DMA,
            pltpu.SemaphoreType.DMA,
            pltpu.SemaphoreType.DMA,
            pltpu.SemaphoreType.DMA,
            pltpu.SemaphoreType.DMA,
            pltpu.SemaphoreType.DMA,
            pltpu.SemaphoreType.DMA,
            pltpu.SemaphoreType.DMA,
        ) + extra,
    )
    def seg_kernel(u_hbm, src_hbm, dst_hbm, *rest):
        if gather_q:
            (q_hbm, g_out, uq_out, acc, stage, sslab, dslab,
             b0, b1, b2, b3, g0, g1, g2, g3, s0, s1, s2, s3,
             qidx, qidx2, qrows) = rest
        else:
            (s_out, acc, stage, sslab, dslab,
             b0, b1, b2, b3, g0, g1, g2, g3, s0, s1, s2, s3) = rest
        bufs = (b0, b1, b2, b3)
        gsem = (g0, g1, g2, g3)
        ssem = (s0, s1, s2, s3)

        c = lax.axis_index("c")
        s = lax.axis_index("s")
        ccol = pl.multiple_of(c * _H, 8)

        base = pl.multiple_of(s * (_NCH * _CHUNK), 8)

        def load_slab(hbm, slab, mul, off):
            def rep_batch(b, carry):
                bo = pl.multiple_of(base + b * _DB, 8)
                pltpu.sync_copy(hbm.at[pl.ds(bo, _DB)], stage)

                def rep_row(j, carry2):
                    for k in range(_CHUNK // 16):
                        slab[b * _DBR + j, pl.ds(k * 16, 16)] = (
                            stage[pl.ds(j * _CHUNK + k * 16, 16)] * mul
                            + off)
                    return carry2
                lax.fori_loop(0, _DBR, rep_row, 0)
                return carry
            lax.fori_loop(0, _NCH // _DBR, rep_batch, 0)

        load_slab(src_hbm, sslab, 2, c)
        load_slab(dst_hbm, dslab, 1, 0)

        def zero_body(r, carry):
            for l in range(_H // 16):
                b0[r, pl.ds(l * 16, 16)] = jnp.zeros((16,), _F32)
            return carry
        lax.fori_loop(0, _CHUNK, zero_body, 0)

        def init(r0, nr):
            for k in range(nr // 64):
                rr = pl.multiple_of(r0 + k * 64, 8)
                pltpu.async_copy(b0.at[pl.ds(0, 64), :],
                                 acc.at[pl.ds(rr, 64), :], g0)
            for k in range(nr // 64):
                pltpu.make_async_copy(b0.at[pl.ds(0, 64), :],
                                      acc.at[pl.ds(0, 64), :], g0).wait()
        _for_stripe(s, init)
        plsc.subcore_barrier()

        def gather(j, buf, sem):
            pltpu.async_copy(u_hbm.at[sslab.at[j]], buf, sem)

        def gwait(buf, sem):
            pltpu.make_async_copy(u_hbm.at[sslab.at[0]], buf, sem).wait()

        def ascat(j, buf, sem):
            pltpu.async_copy(buf, acc.at[dslab.at[j]], sem, add=True)

        def swait(buf, sem):
            pltpu.make_async_copy(buf, acc.at[dslab.at[0]], sem).wait()

        for t in range(4):
            gather(t, bufs[t], gsem[t])

        def edge_body(g, carry):
            j = 4 * g
            for t in range(4):
                gwait(bufs[t], gsem[t])
                ascat(j + t, bufs[t], ssem[t])
            for t in range(4):
                swait(bufs[t], ssem[t])

                @pl.when(j + 4 + t < _NCH)
                def _():
                    gather(j + 4 + t, bufs[t], gsem[t])
            return carry
        lax.fori_loop(0, _NCH // 4, edge_body, 0)
        for t in range(2):
            gwait(bufs[t], gsem[t])
            ascat(_NCH - 2 + t, bufs[t], ssem[t])
        for t in range(2):
            swait(bufs[t], ssem[t])
        plsc.subcore_barrier()

        if gather_q:
            qb = pl.multiple_of(s * _QC, 8)
            for b in range(_QC // _QB):
                qo = pl.multiple_of(qb + b * _QB, 8)
                pltpu.sync_copy(q_hbm.at[pl.ds(qo, _QB)], qidx.at[b])

            def adj(b, carry):
                qidx2[b, pl.ds(0, 16)] = qidx[b, pl.ds(0, 16)] * 2 + c
                return carry
            lax.fori_loop(0, _QC // _QB, adj, 0)
            for b in range(_QC // _QB):
                qo = pl.multiple_of(qb + b * _QB, 8)
                pltpu.async_copy(acc.at[qidx.at[b]], qrows, g0).wait()
                pltpu.sync_copy(
                    qrows, g_out.at[pl.ds(qo, _QB), pl.ds(ccol, _H)])
                pltpu.async_copy(u_hbm.at[qidx2.at[b]], qrows, g0).wait()
                pltpu.sync_copy(
                    qrows, uq_out.at[pl.ds(qo, _QB), pl.ds(ccol, _H)])
        else:
            stb = (b1, b2, b3)
            sma = (g1, g2, g3)
            smh = (s1, s2, s3)

            def wout(r0, nr):
                nb = nr // 64

                def start_a(k, t):
                    rr = pl.multiple_of(r0 + k * 64, 8)
                    pltpu.async_copy(acc.at[pl.ds(rr, 64), :],
                                     stb[t].at[pl.ds(0, 64), :], sma[t])

                def wait_a(t):
                    pltpu.make_async_copy(
                        acc.at[pl.ds(0, 64), :],
                        stb[t].at[pl.ds(0, 64), :], sma[t]).wait()

                def start_h(k, t):
                    rr = pl.multiple_of(r0 + k * 64, 8)
                    pltpu.async_copy(
                        stb[t].at[pl.ds(0, 64), :],
                        s_out.at[pl.ds(rr, 64), pl.ds(ccol, _H)], smh[t])

                def wait_h(t):
                    pltpu.make_async_copy(
                        stb[t].at[pl.ds(0, 64), :],
                        s_out.at[pl.ds(0, 64), pl.ds(ccol, _H)],
                        smh[t]).wait()

                start_a(0, 0)
                for k in range(nb):
                    t = k % 3
                    wait_a(t)
                    start_h(k, t)
                    if k + 1 < nb:
                        tn = (k + 1) % 3
                        if k + 1 >= 3:
                            wait_h(tn)
                        start_a(k + 1, tn)
                for j in range(max(0, nb - 3), nb):
                    wait_h(j % 3)
            _for_stripe(s, wout)

    return seg_kernel


_deg_call = _build_deg()
_seg1_call = _build_seg(gather_q=False)
_seg2_call = _build_seg(gather_q=True)


_BR = 1000


def _tc1(x, w_in, b_in, w1, degp):
    def body(x_ref, win_ref, bin_ref, w1_ref, deg_ref, out_ref):
        h = jnp.maximum(
            jnp.dot(x_ref[...], win_ref[...],
                    preferred_element_type=_F32) + bin_ref[...], 0.0)
        dinv = lax.rsqrt(deg_ref[0] + deg_ref[1] + 1.0)
        out_ref[...] = jnp.dot(h, w1_ref[...],
                               preferred_element_type=_F32) * dinv

    return pl.pallas_call(
        body,
        grid=(_N // _BR,),
        in_specs=[pl.BlockSpec((_BR, _D), lambda i: (i, 0)),
                  pl.BlockSpec((_D, _D), lambda i: (0, 0)),
                  pl.BlockSpec((1, _D), lambda i: (0, 0)),
                  pl.BlockSpec((_D, _D), lambda i: (0, 0)),
                  pl.BlockSpec((_NC, _BR, 1), lambda i: (0, i, 0))],
        out_specs=pl.BlockSpec((_BR, _D), lambda i: (i, 0)),
        out_shape=jax.ShapeDtypeStruct((_N, _D), _F32),
    )(x, w_in, b_in, w1, degp)


def _tc2(sfull, u1, degp, b1, w2):
    def body(s_ref, u1_ref, deg_ref, b1_ref, w2_ref, out_ref):
        dinv = lax.rsqrt(deg_ref[0] + deg_ref[1] + 1.0)
        st = s_ref[...] + u1_ref[...]
        h = jnp.maximum(st * dinv + b1_ref[...], 0.0)
        out_ref[...] = jnp.dot(h, w2_ref[...],
                               preferred_element_type=_F32) * dinv

    return pl.pallas_call(
        body,
        grid=(_N // _BR,),
        in_specs=[pl.BlockSpec((_BR, _D), lambda i: (i, 0)),
                  pl.BlockSpec((_BR, _D), lambda i: (i, 0)),
                  pl.BlockSpec((_NC, _BR, 1), lambda i: (0, i, 0)),
                  pl.BlockSpec((1, _D), lambda i: (0, 0)),
                  pl.BlockSpec((_D, _D), lambda i: (0, 0))],
        out_specs=pl.BlockSpec((_BR, _D), lambda i: (i, 0)),
        out_shape=jax.ShapeDtypeStruct((_N, _D), _F32),
    )(sfull, u1, degp, b1, w2)


def _tc3(g, uq, dqp, b2, w_out, b_out):
    def body(g_ref, uq_ref, dq_ref, b2_ref, wout_ref, bout_ref, out_ref):
        dinv = lax.rsqrt(dq_ref[0] + dq_ref[1] + 1.0)
        st = g_ref[...] + uq_ref[...]
        h = jnp.maximum(st * dinv + b2_ref[...], 0.0)
        out_ref[...] = jnp.dot(h, wout_ref[...],
                               preferred_element_type=_F32) + bout_ref[...]

    return pl.pallas_call(
        body,
        grid=(1,),
        in_specs=[pl.BlockSpec((_Q, _D), lambda i: (0, 0)),
                  pl.BlockSpec((_Q, _D), lambda i: (0, 0)),
                  pl.BlockSpec((_NC, _Q, 1), lambda i: (0, 0, 0)),
                  pl.BlockSpec((1, _D), lambda i: (0, 0)),
                  pl.BlockSpec((_D, _D), lambda i: (0, 0)),
                  pl.BlockSpec((1, _D), lambda i: (0, 0))],
        out_specs=pl.BlockSpec((_Q, _D), lambda i: (0, 0)),
        out_shape=jax.ShapeDtypeStruct((_Q, _D), _F32),
    )(g, uq, dqp, b2, w_out, b_out)


def kernel(x, current_node_idx, edge_index, W_in, b_in, W1, b1, W2, b2,
           W_out, b_out):
    src1 = edge_index[0].astype(jnp.int32)
    dst1 = edge_index[1].astype(jnp.int32)
    q = current_node_idx.astype(jnp.int32)

    degp, dqp = _deg_call(dst1, q)
    degp3 = degp.reshape(_NC, _N, 1)

    u1 = _tc1(x, W_in, b_in.reshape(1, _D), W1, degp3)
    sfull = _seg1_call(u1.reshape(2 * _N, _H), src1, dst1)
    u2 = _tc2(sfull, u1, degp3, b1.reshape(1, _D), W2)
    g, uq = _seg2_call(u2.reshape(2 * _N, _H), src1, dst1, q)
    return _tc3(g, uq, dqp.reshape(_NC, _Q, 1), b2.reshape(1, _D),
                W_out, b_out.reshape(1, _D))


# speedup vs baseline: 29.0091x; 1.0179x over previous
import functools

import jax
import jax.numpy as jnp
from jax import lax
from jax.experimental import pallas as pl
from jax.experimental.pallas import tpu as pltpu
from jax.experimental.pallas import tpu_sc as plsc

_N = 10000
_E = 320000
_D = 128
_H = _D // 2
_Q = 1024

_NC = 2
_NS = 16
_CHUNK = 80
_QC = _Q // _NS

_STRIPE = 640
_LAST_OFF = _N - 512
_LAST = 512

_F32 = jnp.float32


def _mesh():
    return plsc.VectorSubcoreMesh(
        core_axis_name="c", subcore_axis_name="s",
        num_cores=_NC, num_subcores=_NS)


def _for_stripe(s, emit):
    @pl.when(s < _NS - 1)
    def _():
        emit(pl.multiple_of(s * _STRIPE, 8), _STRIPE)

    @pl.when(s == _NS - 1)
    def _():
        emit(_LAST_OFF, _LAST)


def _build_deg():
    ndch = _E // _NC // _NS // _CHUNK

    @functools.partial(
        pl.kernel,
        out_type=(jax.ShapeDtypeStruct((_NC * _N,), _F32),
                  jax.ShapeDtypeStruct((_NC * _Q,), _F32)),
        mesh=_mesh(),
        scratch_types=(
            pltpu.VMEM_SHARED((_N,), _F32),
            pltpu.VMEM((_STRIPE,), _F32),
            pltpu.VMEM((ndch * _CHUNK,), jnp.int32),
            pltpu.VMEM((ndch, _CHUNK), jnp.int32),
            pltpu.VMEM((_CHUNK,), _F32),
            pltpu.VMEM((1, _QC), jnp.int32),
            pltpu.VMEM((_QC,), _F32),
            pltpu.SemaphoreType.DMA,
        ),
    )
    def deg_kernel(dst_hbm, q_hbm, deg_out, dq_out,
                   deg_sp, stage, dslab1, dslab, ones, qidx, dqv, sem):
        c = lax.axis_index("c")
        s = lax.axis_index("s")

        def zero_body(k, carry):
            stage[pl.ds(k * 16, 16)] = jnp.zeros((16,), _F32)
            return carry
        lax.fori_loop(0, _STRIPE // 16, zero_body, 0)
        for k in range(_CHUNK // 16):
            ones[pl.ds(k * 16, 16)] = jnp.full((16,), 1.0, _F32)

        base = pl.multiple_of((c * _NS + s) * (ndch * _CHUNK), 8)
        pltpu.sync_copy(dst_hbm.at[pl.ds(base, ndch * _CHUNK)], dslab1)

        def repack(j, carry):
            for k in range(_CHUNK // 16):
                dslab[j, pl.ds(k * 16, 16)] = (
                    dslab1[pl.ds(j * _CHUNK + k * 16, 16)])
            return carry
        lax.fori_loop(0, ndch, repack, 0)

        def init(r0, nr):
            pltpu.sync_copy(stage.at[pl.ds(0, nr)], deg_sp.at[pl.ds(r0, nr)])
        _for_stripe(s, init)
        plsc.subcore_barrier()

        def edge_body(r, carry):
            for t in range(5):
                pltpu.async_copy(ones, deg_sp.at[dslab.at[r * 5 + t]],
                                 sem, add=True)
            for t in range(5):
                pltpu.make_async_copy(ones, deg_sp.at[dslab.at[0]],
                                      sem).wait()
            return carry
        lax.fori_loop(0, ndch // 5, edge_body, 0)
        plsc.subcore_barrier()

        def wout(r0, nr):
            pltpu.sync_copy(deg_sp.at[pl.ds(r0, nr)], stage.at[pl.ds(0, nr)])
            o0 = pl.multiple_of(c * _N + r0, 8)
            pltpu.sync_copy(stage.at[pl.ds(0, nr)], deg_out.at[pl.ds(o0, nr)])
        _for_stripe(s, wout)
        qb = pl.multiple_of(s * _QC, 8)
        pltpu.sync_copy(q_hbm.at[pl.ds(qb, _QC)], qidx.at[0])
        pltpu.async_copy(deg_sp.at[qidx.at[0]], dqv, sem).wait()
        oq = pl.multiple_of(c * _Q + qb, 8)
        pltpu.sync_copy(dqv, dq_out.at[pl.ds(oq, _QC)])

    return deg_kernel


_NCH = _E // _NS // _CHUNK
_DB = 2000
_DBR = _DB // _CHUNK
_QB = 16


def _build_seg(gather_q):
    if gather_q:
        out_type = (jax.ShapeDtypeStruct((_Q, _D), _F32),
                    jax.ShapeDtypeStruct((_Q, _D), _F32))
        extra = (pltpu.VMEM((_QC // _QB, _QB), jnp.int32),
                 pltpu.VMEM((_QC // _QB, _QB), jnp.int32),
                 pltpu.VMEM((_QB, _H), _F32))
    else:
        out_type = jax.ShapeDtypeStruct((_N, _D), _F32)
        extra = ()

    @functools.partial(
        pl.kernel,
        out_type=out_type,
        mesh=_mesh(),
        compiler_params=pltpu.CompilerParams(use_tc_tiling_on_sc=False),
        scratch_types=(
            pltpu.VMEM_SHARED((_N, _H), _F32),
            pltpu.VMEM((_DB,), jnp.int32),
            pltpu.VMEM((_NCH, _CHUNK), jnp.int32),
            pltpu.VMEM((_NCH, _CHUNK), jnp.int32),
            pltpu.VMEM((_CHUNK, _H), _F32),
            pltpu.VMEM((_CHUNK, _H), _F32),
            pltpu.VMEM((_CHUNK, _H), _F32),
            pltpu.VMEM((_CHUNK, _H), _F32),
            pltpu.SemaphoreType.DMA,
            pltpu.SemaphoreType.DMA,
            pltpu.SemaphoreType.DMA,
            pltpu.SemaphoreType.DMA,
            pltpu.SemaphoreType.DMA,
            pltpu.SemaphoreType.DMA,
            pltpu.SemaphoreType.DMA,
            pltpu.SemaphoreType.DMA,
        ) + extra,
    )
    def seg_kernel(u_hbm, src_hbm, dst_hbm, *rest):
        if gather_q:
            (q_hbm, g_out, uq_out, acc, stage, sslab, dslab,
             b0, b1, b2, b3, g0, g1, g2, g3, s0, s1, s2, s3,
             qidx, qidx2, qrows) = rest
        else:
            (s_out, acc, stage, sslab, dslab,
             b0, b1, b2, b3, g0, g1, g2, g3, s0, s1, s2, s3) = rest
        bufs = (b0, b1, b2, b3)
        gsem = (g0, g1, g2, g3)
        ssem = (s0, s1, s2, s3)

        c = lax.axis_index("c")
        s = lax.axis_index("s")
        ccol = pl.multiple_of(c * _H, 8)

        base = pl.multiple_of(s * (_NCH * _CHUNK), 8)

        def load_slab(hbm, slab, mul, off):
            def rep_batch(b, carry):
                bo = pl.multiple_of(base + b * _DB, 8)
                pltpu.sync_copy(hbm.at[pl.ds(bo, _DB)], stage)

                def rep_row(j, carry2):
                    for k in range(_CHUNK // 16):
                        slab[b * _DBR + j, pl.ds(k * 16, 16)] = (
                            stage[pl.ds(j * _CHUNK + k * 16, 16)] * mul
                            + off)
                    return carry2
                lax.fori_loop(0, _DBR, rep_row, 0)
                return carry
            lax.fori_loop(0, _NCH // _DBR, rep_batch, 0)

        load_slab(src_hbm, sslab, 2, c)
        load_slab(dst_hbm, dslab, 1, 0)

        def zero_body(r, carry):
            for l in range(_H // 16):
                b0[r, pl.ds(l * 16, 16)] = jnp.zeros((16,), _F32)
            return carry
        lax.fori_loop(0, _CHUNK, zero_body, 0)

        def init(r0, nr):
            for k in range(nr // 64):
                rr = pl.multiple_of(r0 + k * 64, 8)
                pltpu.async_copy(b0.at[pl.ds(0, 64), :],
                                 acc.at[pl.ds(rr, 64), :], g0)
            for k in range(nr // 64):
                pltpu.make_async_copy(b0.at[pl.ds(0, 64), :],
                                      acc.at[pl.ds(0, 64), :], g0).wait()
        _for_stripe(s, init)
        plsc.subcore_barrier()

        def gather(j, buf, sem):
            pltpu.async_copy(u_hbm.at[sslab.at[j]], buf, sem)

        def gwait(buf, sem):
            pltpu.make_async_copy(u_hbm.at[sslab.at[0]], buf, sem).wait()

        def ascat(j, buf, sem):
            pltpu.async_copy(buf, acc.at[dslab.at[j]], sem, add=True)

        def swait(buf, sem):
            pltpu.make_async_copy(buf, acc.at[dslab.at[0]], sem).wait()

        for t in range(4):
            gather(t, bufs[t], gsem[t])

        def edge_body(g, carry):
            j = 4 * g
            for t in range(4):
                gwait(bufs[t], gsem[t])
                ascat(j + t, bufs[t], ssem[t])
            for t in range(4):
                swait(bufs[t], ssem[t])

                @pl.when(j + 4 + t < _NCH)
                def _():
                    gather(j + 4 + t, bufs[t], gsem[t])
            return carry
        lax.fori_loop(0, _NCH // 4, edge_body, 0)
        for t in range(2):
            gwait(bufs[t], gsem[t])
            ascat(_NCH - 2 + t, bufs[t], ssem[t])
        for t in range(2):
            swait(bufs[t], ssem[t])
        plsc.subcore_barrier()

        if gather_q:
            qb = pl.multiple_of(s * _QC, 8)
            for b in range(_QC // _QB):
                qo = pl.multiple_of(qb + b * _QB, 8)
                pltpu.sync_copy(q_hbm.at[pl.ds(qo, _QB)], qidx.at[b])

            def adj(b, carry):
                qidx2[b, pl.ds(0, 16)] = qidx[b, pl.ds(0, 16)] * 2 + c
                return carry
            lax.fori_loop(0, _QC // _QB, adj, 0)
            for b in range(_QC // _QB):
                qo = pl.multiple_of(qb + b * _QB, 8)
                pltpu.async_copy(acc.at[qidx.at[b]], qrows, g0).wait()
                pltpu.sync_copy(
                    qrows, g_out.at[pl.ds(qo, _QB), pl.ds(ccol, _H)])
                pltpu.async_copy(u_hbm.at[qidx2.at[b]], qrows, g0).wait()
                pltpu.sync_copy(
                    qrows, uq_out.at[pl.ds(qo, _QB), pl.ds(ccol, _H)])
        else:
            stb = (b1, b2, b3)
            sma = (g1, g2, g3)
            smh = (s1, s2, s3)

            def wout(r0, nr):
                nb = nr // 64

                def start_a(k, t):
                    rr = pl.multiple_of(r0 + k * 64, 8)
                    pltpu.async_copy(acc.at[pl.ds(rr, 64), :],
                                     stb[t].at[pl.ds(0, 64), :], sma[t])

                def wait_a(t):
                    pltpu.make_async_copy(
                        acc.at[pl.ds(0, 64), :],
                        stb[t].at[pl.ds(0, 64), :], sma[t]).wait()

                def start_h(k, t):
                    rr = pl.multiple_of(r0 + k * 64, 8)
                    pltpu.async_copy(
                        stb[t].at[pl.ds(0, 64), :],
                        s_out.at[pl.ds(rr, 64), pl.ds(ccol, _H)], smh[t])

                def wait_h(t):
                    pltpu.make_async_copy(
                        stb[t].at[pl.ds(0, 64), :],
                        s_out.at[pl.ds(0, 64), pl.ds(ccol, _H)],
                        smh[t]).wait()

                start_a(0, 0)
                for k in range(nb):
                    t = k % 3
                    wait_a(t)
                    start_h(k, t)
                    if k + 1 < nb:
                        tn = (k + 1) % 3
                        if k + 1 >= 3:
                            wait_h(tn)
                        start_a(k + 1, tn)
                for j in range(max(0, nb - 3), nb):
                    wait_h(j % 3)
            _for_stripe(s, wout)

    return seg_kernel


_deg_call = _build_deg()
_seg1_call = _build_seg(gather_q=False)
_seg2_call = _build_seg(gather_q=True)


_BR = 1000


def _tc1(x, w_in, b_in, w1, degp):
    def body(x_ref, win_ref, bin_ref, w1_ref, deg_ref, out_ref):
        h = jnp.maximum(
            jnp.dot(x_ref[...], win_ref[...],
                    preferred_element_type=_F32) + bin_ref[...], 0.0)
        dinv = lax.rsqrt(deg_ref[0] + deg_ref[1] + 1.0)
        out_ref[...] = jnp.dot(h, w1_ref[...],
                               preferred_element_type=_F32) * dinv

    return pl.pallas_call(
        body,
        grid=(_N // _BR,),
        in_specs=[pl.BlockSpec((_BR, _D), lambda i: (i, 0)),
                  pl.BlockSpec((_D, _D), lambda i: (0, 0)),
                  pl.BlockSpec((1, _D), lambda i: (0, 0)),
                  pl.BlockSpec((_D, _D), lambda i: (0, 0)),
                  pl.BlockSpec((_NC, _BR, 1), lambda i: (0, i, 0))],
        out_specs=pl.BlockSpec((_BR, _D), lambda i: (i, 0)),
        out_shape=jax.ShapeDtypeStruct((_N, _D), _F32),
    )(x, w_in, b_in, w1, degp)


def _tc2(sfull, u1, degp, b1, w2):
    def body(s_ref, u1_ref, deg_ref, b1_ref, w2_ref, out_ref):
        dinv = lax.rsqrt(deg_ref[0] + deg_ref[1] + 1.0)
        st = s_ref[...] + u1_ref[...]
        h = jnp.maximum(st * dinv + b1_ref[...], 0.0)
        out_ref[...] = jnp.dot(h, w2_ref[...],
                               preferred_element_type=_F32) * dinv

    return pl.pallas_call(
        body,
        grid=(_N // _BR,),
        in_specs=[pl.BlockSpec((_BR, _D), lambda i: (i, 0)),
                  pl.BlockSpec((_BR, _D), lambda i: (i, 0)),
                  pl.BlockSpec((_NC, _BR, 1), lambda i: (0, i, 0)),
                  pl.BlockSpec((1, _D), lambda i: (0, 0)),
                  pl.BlockSpec((_D, _D), lambda i: (0, 0))],
        out_specs=pl.BlockSpec((_BR, _D), lambda i: (i, 0)),
        out_shape=jax.ShapeDtypeStruct((_N, _D), _F32),
    )(sfull, u1, degp, b1, w2)


def _tc3(g, uq, dqp, b2, w_out, b_out):
    def body(g_ref, uq_ref, dq_ref, b2_ref, wout_ref, bout_ref, out_ref):
        dinv = lax.rsqrt(dq_ref[0] + dq_ref[1] + 1.0)
        st = g_ref[...] + uq_ref[...]
        h = jnp.maximum(st * dinv + b2_ref[...], 0.0)
        out_ref[...] = jnp.dot(h, wout_ref[...],
                               preferred_element_type=_F32) + bout_ref[...]

    return pl.pallas_call(
        body,
        grid=(1,),
        in_specs=[pl.BlockSpec((_Q, _D), lambda i: (0, 0)),
                  pl.BlockSpec((_Q, _D), lambda i: (0, 0)),
                  pl.BlockSpec((_NC, _Q, 1), lambda i: (0, 0, 0)),
                  pl.BlockSpec((1, _D), lambda i: (0, 0)),
                  pl.BlockSpec((_D, _D), lambda i: (0, 0)),
                  pl.BlockSpec((1, _D), lambda i: (0, 0))],
        out_specs=pl.BlockSpec((_Q, _D), lambda i: (0, 0)),
        out_shape=jax.ShapeDtypeStruct((_Q, _D), _F32),
    )(g, uq, dqp, b2, w_out, b_out)


def kernel(x, current_node_idx, edge_index, W_in, b_in, W1, b1, W2, b2,
           W_out, b_out):
    src1 = edge_index[0].astype(jnp.int32)
    dst1 = edge_index[1].astype(jnp.int32)
    q = current_node_idx.astype(jnp.int32)

    degp, dqp = _deg_call(dst1, q)
    degp3 = degp.reshape(_NC, _N, 1)

    u1 = _tc1(x, W_in, b_in.reshape(1, _D), W1, degp3)
    sfull = _seg1_call(u1.reshape(2 * _N, _H), src1, dst1)
    u2 = _tc2(sfull, u1, degp3, b1.reshape(1, _D), W2)
    g, uq = _seg2_call(u2.reshape(2 * _N, _H), src1, dst1, q)
    return _tc3(g, uq, dqp.reshape(_NC, _Q, 1), b2.reshape(1, _D),
                W_out, b_out.reshape(1, _D))
